# Initial kernel scaffold; baseline (speedup 1.0000x reference)
import jax
import jax.numpy as jnp
from jax.experimental import pallas as pl


def kernel(texts, table, W, b, gamma, beta):
    raise NotImplementedError("write your pallas kernel here")


# speedup vs baseline: 1.4909x; 1.4909x over previous
import functools

import jax
import jax.numpy as jnp
from jax import lax
from jax.experimental import pallas as pl
from jax.experimental.pallas import tpu as pltpu
from jax.experimental.pallas import tpu_sc as plsc

VOCAB = 100000
EMBED = 128
PROJ = 512
LN_EPS = 1e-5

NC = 2
NS = 16
NW = NC * NS

ROWS_PER_BLOCK = 2000


def _project_ln_body(table_ref, w_ref, b_ref, gamma_ref, beta_ref, out_ref):
    h = lax.dot_general(
        table_ref[...], w_ref[...],
        dimension_numbers=(((1,), (1,)), ((), ())),
        preferred_element_type=jnp.float32,
    )
    h = h + b_ref[...]
    mu = jnp.mean(h, axis=-1, keepdims=True)
    var = jnp.mean((h - mu) ** 2, axis=-1, keepdims=True)
    out_ref[...] = (h - mu) * lax.rsqrt(var + LN_EPS) * gamma_ref[...] + beta_ref[...]


def _project_ln(table, W, b, gamma, beta):
    grid = VOCAB // ROWS_PER_BLOCK
    return pl.pallas_call(
        _project_ln_body,
        grid=(grid,),
        in_specs=[
            pl.BlockSpec((ROWS_PER_BLOCK, EMBED), lambda i: (i, 0)),
            pl.BlockSpec((PROJ, EMBED), lambda i: (0, 0)),
            pl.BlockSpec((1, PROJ), lambda i: (0, 0)),
            pl.BlockSpec((1, PROJ), lambda i: (0, 0)),
            pl.BlockSpec((1, PROJ), lambda i: (0, 0)),
        ],
        out_specs=pl.BlockSpec((ROWS_PER_BLOCK, PROJ), lambda i: (i, 0)),
        out_shape=jax.ShapeDtypeStruct((VOCAB, PROJ), jnp.float32),
    )(table, W, b.reshape(1, PROJ), gamma.reshape(1, PROJ), beta.reshape(1, PROJ))


def _make_sc_gather(n_tokens, chunk):
    n_chunks = n_tokens // (NW * chunk)
    per_w = n_chunks * chunk
    mesh = plsc.VectorSubcoreMesh(core_axis_name="c", subcore_axis_name="s")

    @functools.partial(
        pl.kernel,
        out_type=jax.ShapeDtypeStruct((n_tokens, PROJ), jnp.float32),
        mesh=mesh,
        scratch_types=[
            pltpu.VMEM((n_chunks, chunk), jnp.int32),
            pltpu.VMEM((chunk, PROJ), jnp.float32),
            pltpu.SemaphoreType.DMA,
        ],
    )
    def gather_kernel(f_hbm, idx_hbm, out_hbm, idx_v, rows_v, sem):
        wid = lax.axis_index("s") * NC + lax.axis_index("c")
        pltpu.sync_copy(idx_hbm.at[wid], idx_v)
        base = wid * per_w

        def step(j, carry):
            pltpu.async_copy(f_hbm.at[idx_v.at[j]], rows_v, sem).wait()
            pltpu.sync_copy(rows_v, out_hbm.at[pl.ds(base + j * chunk, chunk)])
            return carry

        lax.fori_loop(0, n_chunks, step, 0)

    return gather_kernel


def kernel(texts, table, W, b, gamma, beta):
    B, L = texts.shape
    n_tokens = B * L
    chunk = 128
    f = _project_ln(table, W, b, gamma, beta)
    idx = texts.reshape(NW, n_tokens // (NW * chunk), chunk).astype(jnp.int32)
    out = _make_sc_gather(n_tokens, chunk)(f, idx)
    return out.reshape(B, L, PROJ)
